# baseline (speedup 1.0000x reference)
import functools

import jax
import jax.numpy as jnp
from jax import lax
from jax.experimental import pallas as pl
from jax.experimental.pallas import tpu as pltpu
from jax.experimental.pallas import tpu_sc as plsc

N = 10000
E = 320000
D = 128
NC = 2
NS = 16
NW = NC * NS
BK = 128
SB = 16
NSB = 5
NB = SB * NSB
EPT = NB * BK
NSB0 = 10
NSB1 = 0
W0 = NSB0 * SB * BK
W1 = NSB1 * SB * BK
ETOT = NS * (W0 + W1)
ACC = NS * 640
RPT = ACC // NS
DUMMY = N

_mesh = plsc.VectorSubcoreMesh(
    core_axis_name="c", subcore_axis_name="s", num_cores=NC, num_subcores=NS)


@functools.partial(
    pl.kernel,
    out_type=jax.ShapeDtypeStruct((NC, ACC, D), jnp.float32),
    mesh=_mesh,
    scratch_types=[
        pltpu.VMEM((NB, BK), jnp.int32),
        pltpu.VMEM((BK, D), jnp.float32),
        pltpu.VMEM_SHARED((ACC, D), jnp.float32),
    ],
)
def _deg_kernel(dst_hbm, ones_hbm, zrow_hbm, out_hbm, dst_v, buf, dacc):
    c = lax.axis_index("c")
    s = lax.axis_index("s")
    wid = c * NS + s
    pltpu.sync_copy(dst_hbm.at[wid], dst_v)
    pltpu.sync_copy(zrow_hbm, buf)
    for k in range(RPT // BK):
        pltpu.sync_copy(buf, dacc.at[pl.ds(s * RPT + k * BK, BK)])
    plsc.subcore_barrier()
    pltpu.sync_copy(ones_hbm, buf)

    def body(p, carry):
        pltpu.sync_copy(buf, dacc.at[dst_v.at[p]], add=True)
        return carry

    lax.fori_loop(0, NB, body, 0)
    plsc.subcore_barrier()
    for k in range(RPT // BK):
        pltpu.sync_copy(dacc.at[pl.ds(s * RPT + k * BK, BK)], buf)
        pltpu.sync_copy(buf, out_hbm.at[c, pl.ds(s * RPT + k * BK, BK)])


@functools.partial(
    pl.kernel,
    out_type=jax.ShapeDtypeStruct((NC, ACC, D), jnp.float32),
    mesh=_mesh,
    scratch_types=[
        pltpu.VMEM((SB * BK,), jnp.int32),
        pltpu.VMEM((SB, BK), jnp.int32),
        pltpu.VMEM((BK, D), jnp.float32),
        pltpu.VMEM((BK, D), jnp.float32),
        pltpu.VMEM_SHARED((ACC, D), jnp.float32),
        pltpu.SemaphoreType.DMA,
        pltpu.SemaphoreType.DMA,
        pltpu.SemaphoreType.DMA,
    ],
)
def _agg_kernel(g_hbm, src_hbm, dst_hbm, zrow_hbm, out_hbm,
                src_v, dst_v, r0, r1, acc, g0, g1, ssem):
    c = lax.axis_index("c")
    s = lax.axis_index("s")
    base_e = jnp.where(c == 0, s * W0, NS * W0 + s * W1)
    nsb = jnp.where(c == 0, NSB0, NSB1)

    pltpu.sync_copy(zrow_hbm, r0)
    for k in range(RPT // BK):
        pltpu.sync_copy(r0, acc.at[pl.ds(s * RPT + k * BK, BK)])
    plsc.subcore_barrier()

    def outer(o, carry):
        eo = pl.multiple_of(base_e + o * (SB * BK), SB * BK)
        pltpu.sync_copy(src_hbm.at[pl.ds(eo, SB * BK)], src_v)
        pltpu.sync_copy(dst_hbm.at[pl.ds(pl.multiple_of(eo // BK, SB), SB)], dst_v)

        def inner(p, carry2):
            base = p * (2 * BK)
            c0 = pltpu.async_copy(
                g_hbm.at[src_v.at[pl.ds(base, BK)]], r0, g0)
            c1 = pltpu.async_copy(
                g_hbm.at[src_v.at[pl.ds(base + BK, BK)]], r1, g1)
            c0.wait()
            s0 = pltpu.async_copy(r0, acc.at[dst_v.at[2 * p]], ssem, add=True)
            c1.wait()
            s1 = pltpu.async_copy(r1, acc.at[dst_v.at[2 * p + 1]], ssem, add=True)
            s0.wait()
            s1.wait()
            return carry2

        lax.fori_loop(0, SB // 2, inner, 0)
        return carry

    lax.fori_loop(0, nsb, outer, 0)
    plsc.subcore_barrier()

    for k in range(RPT // BK):
        pltpu.sync_copy(acc.at[pl.ds(s * RPT + k * BK, BK)], r0)
        pltpu.sync_copy(r0, out_hbm.at[c, pl.ds(s * RPT + k * BK, BK)])


def _lin_body(deg_ref, x_ref, w_ref, g_ref, dis_ref):
    deg = deg_ref[0, :N, 0:1] + deg_ref[1, :N, 0:1] + 1.0
    dis = lax.rsqrt(deg)
    dis_ref[...] = dis
    h = jnp.dot(x_ref[...], w_ref[...], preferred_element_type=jnp.float32)
    g_ref[...] = h * dis


_lin_call = pl.pallas_call(
    _lin_body,
    out_shape=(jax.ShapeDtypeStruct((N, D), jnp.float32),
               jax.ShapeDtypeStruct((N, 1), jnp.float32)))


def _mid_body(acc_ref, g_ref, dis_ref, b_ref, w_ref, out_ref):
    agg = acc_ref[0, :N, :] + acc_ref[1, :N, :] + g_ref[...]
    h = jax.nn.relu(agg * dis_ref[...] + b_ref[...])
    hw = jnp.dot(h, w_ref[...], preferred_element_type=jnp.float32)
    out_ref[...] = hw * dis_ref[...]


_mid_call = pl.pallas_call(
    _mid_body, out_shape=jax.ShapeDtypeStruct((N, D), jnp.float32))


def _final_body(acc_ref, g_ref, dis_ref, b_ref, out_ref):
    agg = acc_ref[0, :N, :] + acc_ref[1, :N, :] + g_ref[...]
    out_ref[...] = jax.nn.relu(agg * dis_ref[...] + b_ref[...])


_final_call = pl.pallas_call(
    _final_body, out_shape=jax.ShapeDtypeStruct((N, D), jnp.float32))


def kernel(x, edge_index, W1, b1, W2, b2, W3, b3):
    src = edge_index[0]
    dst = edge_index[1]
    pad = ETOT - E
    src_p = jnp.concatenate([src, jnp.zeros((pad,), jnp.int32)])
    dst_p = jnp.concatenate([dst, jnp.full((pad,), DUMMY, jnp.int32)])
    dst_deg = dst_p.reshape(NW, NB, BK)
    dst_agg = dst_p.reshape(ETOT // BK, BK)

    zrow = jnp.zeros((BK, D), jnp.float32)
    ones128 = jnp.ones((BK, D), jnp.float32)

    deg_parts = _deg_kernel(dst_deg, ones128, zrow)
    g1, dis_col = _lin_call(deg_parts, x, W1)

    a1 = _agg_kernel(g1, src_p, dst_agg, zrow)
    g2 = _mid_call(a1, g1, dis_col, b1.reshape(1, D), W2)
    a2 = _agg_kernel(g2, src_p, dst_agg, zrow)
    g3 = _mid_call(a2, g2, dis_col, b2.reshape(1, D), W3)
    a3 = _agg_kernel(g3, src_p, dst_agg, zrow)
    return _final_call(a3, g3, dis_col, b3.reshape(1, D))


# speedup vs baseline: 1.5922x; 1.5922x over previous
import functools

import jax
import jax.numpy as jnp
from jax import lax
from jax.experimental import pallas as pl
from jax.experimental.pallas import tpu as pltpu
from jax.experimental.pallas import tpu_sc as plsc

N = 10000
E = 320000
D = 128
NC = 2
NS = 16
NW = NC * NS
BK = 128
SB = 16
NSB = 5
NB = SB * NSB
EPT = NB * BK
NSB0 = 9
NSB1 = 1
W0 = NSB0 * SB * BK
W1 = NSB1 * SB * BK
ETOT = NS * (W0 + W1)
ACC = NS * 640
RPT = ACC // NS
DUMMY = N

_mesh = plsc.VectorSubcoreMesh(
    core_axis_name="c", subcore_axis_name="s", num_cores=NC, num_subcores=NS)


@functools.partial(
    pl.kernel,
    out_type=jax.ShapeDtypeStruct((NC, ACC, D), jnp.float32),
    mesh=_mesh,
    scratch_types=[
        pltpu.VMEM((NB, BK), jnp.int32),
        pltpu.VMEM((BK, D), jnp.float32),
        pltpu.VMEM_SHARED((ACC, D), jnp.float32),
    ],
)
def _deg_kernel(dst_hbm, ones_hbm, zrow_hbm, out_hbm, dst_v, buf, dacc):
    c = lax.axis_index("c")
    s = lax.axis_index("s")
    wid = c * NS + s
    pltpu.sync_copy(dst_hbm.at[wid], dst_v)
    pltpu.sync_copy(zrow_hbm, buf)
    for k in range(RPT // BK):
        pltpu.sync_copy(buf, dacc.at[pl.ds(s * RPT + k * BK, BK)])
    plsc.subcore_barrier()
    pltpu.sync_copy(ones_hbm, buf)

    def body(p, carry):
        pltpu.sync_copy(buf, dacc.at[dst_v.at[p]], add=True)
        return carry

    lax.fori_loop(0, NB, body, 0)
    plsc.subcore_barrier()
    for k in range(RPT // BK):
        pltpu.sync_copy(dacc.at[pl.ds(s * RPT + k * BK, BK)], buf)
        pltpu.sync_copy(buf, out_hbm.at[c, pl.ds(s * RPT + k * BK, BK)])


@functools.partial(
    pl.kernel,
    out_type=jax.ShapeDtypeStruct((NC, ACC, D), jnp.float32),
    mesh=_mesh,
    scratch_types=[
        pltpu.VMEM((SB * BK,), jnp.int32),
        pltpu.VMEM((SB, BK), jnp.int32),
        pltpu.VMEM((BK, D), jnp.float32),
        pltpu.VMEM((BK, D), jnp.float32),
        pltpu.VMEM_SHARED((ACC, D), jnp.float32),
        pltpu.SemaphoreType.DMA,
        pltpu.SemaphoreType.DMA,
        pltpu.SemaphoreType.DMA,
    ],
)
def _agg_kernel(g_hbm, src_hbm, dst_hbm, zrow_hbm, out_hbm,
                src_v, dst_v, r0, r1, acc, g0, g1, ssem):
    c = lax.axis_index("c")
    s = lax.axis_index("s")
    base_e = jnp.where(c == 0, s * W0, NS * W0 + s * W1)
    nsb = jnp.where(c == 0, NSB0, NSB1)

    pltpu.sync_copy(zrow_hbm, r0)
    for k in range(RPT // BK):
        pltpu.sync_copy(r0, acc.at[pl.ds(s * RPT + k * BK, BK)])
    plsc.subcore_barrier()

    def outer(o, carry):
        eo = pl.multiple_of(base_e + o * (SB * BK), SB * BK)
        pltpu.sync_copy(src_hbm.at[pl.ds(eo, SB * BK)], src_v)
        pltpu.sync_copy(dst_hbm.at[pl.ds(pl.multiple_of(eo // BK, SB), SB)], dst_v)

        def inner(p, carry2):
            base = p * (2 * BK)
            c0 = pltpu.async_copy(
                g_hbm.at[src_v.at[pl.ds(base, BK)]], r0, g0)
            c1 = pltpu.async_copy(
                g_hbm.at[src_v.at[pl.ds(base + BK, BK)]], r1, g1)
            c0.wait()
            s0 = pltpu.async_copy(r0, acc.at[dst_v.at[2 * p]], ssem, add=True)
            c1.wait()
            s1 = pltpu.async_copy(r1, acc.at[dst_v.at[2 * p + 1]], ssem, add=True)
            s0.wait()
            s1.wait()
            return carry2

        lax.fori_loop(0, SB // 2, inner, 0)
        return carry

    lax.fori_loop(0, nsb, outer, 0)
    plsc.subcore_barrier()

    for k in range(RPT // BK):
        pltpu.sync_copy(acc.at[pl.ds(s * RPT + k * BK, BK)], r0)
        pltpu.sync_copy(r0, out_hbm.at[c, pl.ds(s * RPT + k * BK, BK)])


def _lin_body(deg_ref, x_ref, w_ref, g_ref, dis_ref):
    deg = deg_ref[0, :N, 0:1] + deg_ref[1, :N, 0:1] + 1.0
    dis = lax.rsqrt(deg)
    dis_ref[...] = dis
    h = jnp.dot(x_ref[...], w_ref[...], preferred_element_type=jnp.float32)
    g_ref[...] = h * dis


_lin_call = pl.pallas_call(
    _lin_body,
    out_shape=(jax.ShapeDtypeStruct((N, D), jnp.float32),
               jax.ShapeDtypeStruct((N, 1), jnp.float32)))


def _mid_body(acc_ref, g_ref, dis_ref, b_ref, w_ref, out_ref):
    agg = acc_ref[0, :N, :] + acc_ref[1, :N, :] + g_ref[...]
    h = jax.nn.relu(agg * dis_ref[...] + b_ref[...])
    hw = jnp.dot(h, w_ref[...], preferred_element_type=jnp.float32)
    out_ref[...] = hw * dis_ref[...]


_mid_call = pl.pallas_call(
    _mid_body, out_shape=jax.ShapeDtypeStruct((N, D), jnp.float32))


def _final_body(acc_ref, g_ref, dis_ref, b_ref, out_ref):
    agg = acc_ref[0, :N, :] + acc_ref[1, :N, :] + g_ref[...]
    out_ref[...] = jax.nn.relu(agg * dis_ref[...] + b_ref[...])


_final_call = pl.pallas_call(
    _final_body, out_shape=jax.ShapeDtypeStruct((N, D), jnp.float32))


def kernel(x, edge_index, W1, b1, W2, b2, W3, b3):
    src = edge_index[0]
    dst = edge_index[1]
    pad = ETOT - E
    src_p = jnp.concatenate([src, jnp.zeros((pad,), jnp.int32)])
    dst_p = jnp.concatenate([dst, jnp.full((pad,), DUMMY, jnp.int32)])
    dst_deg = dst_p.reshape(NW, NB, BK)
    dst_agg = dst_p.reshape(ETOT // BK, BK)

    zrow = jnp.zeros((BK, D), jnp.float32)
    ones128 = jnp.ones((BK, D), jnp.float32)

    deg_parts = _deg_kernel(dst_deg, ones128, zrow)
    g1, dis_col = _lin_call(deg_parts, x, W1)

    a1 = _agg_kernel(g1, src_p, dst_agg, zrow)
    g2 = _mid_call(a1, g1, dis_col, b1.reshape(1, D), W2)
    a2 = _agg_kernel(g2, src_p, dst_agg, zrow)
    g3 = _mid_call(a2, g2, dis_col, b2.reshape(1, D), W3)
    a3 = _agg_kernel(g3, src_p, dst_agg, zrow)
    return _final_call(a3, g3, dis_col, b3.reshape(1, D))


# speedup vs baseline: 2.1829x; 1.3710x over previous
import functools

import jax
import jax.numpy as jnp
from jax import lax
from jax.experimental import pallas as pl
from jax.experimental.pallas import tpu as pltpu
from jax.experimental.pallas import tpu_sc as plsc

N = 10000
E = 320000
D = 128
NC = 2
NS = 16
NW = NC * NS
BK = 128
SB = 16
NSB = 5
NB = SB * NSB
EPT = NB * BK
NSB0 = 9
NSB1 = 1
W0 = NSB0 * SB * BK
W1 = NSB1 * SB * BK
ETOT = NS * (W0 + W1)
ACC = NS * 640
RPT = ACC // NS
DUMMY = N

_mesh = plsc.VectorSubcoreMesh(
    core_axis_name="c", subcore_axis_name="s", num_cores=NC, num_subcores=NS)


@functools.partial(
    pl.kernel,
    out_type=jax.ShapeDtypeStruct((NC, ACC, D), jnp.float32),
    mesh=_mesh,
    scratch_types=[
        pltpu.VMEM((NB, BK), jnp.int32),
        pltpu.VMEM((BK, D), jnp.float32),
        pltpu.VMEM_SHARED((ACC, D), jnp.float32),
    ],
)
def _deg_kernel(dst_hbm, ones_hbm, zrow_hbm, out_hbm, dst_v, buf, dacc):
    c = lax.axis_index("c")
    s = lax.axis_index("s")
    wid = c * NS + s
    pltpu.sync_copy(dst_hbm.at[wid], dst_v)
    pltpu.sync_copy(zrow_hbm, buf)
    for k in range(RPT // BK):
        pltpu.sync_copy(buf, dacc.at[pl.ds(s * RPT + k * BK, BK)])
    plsc.subcore_barrier()
    pltpu.sync_copy(ones_hbm, buf)

    def body(p, carry):
        pltpu.sync_copy(buf, dacc.at[dst_v.at[p]], add=True)
        return carry

    lax.fori_loop(0, NB, body, 0)
    plsc.subcore_barrier()
    for k in range(RPT // BK):
        pltpu.sync_copy(dacc.at[pl.ds(s * RPT + k * BK, BK)], buf)
        pltpu.sync_copy(buf, out_hbm.at[c, pl.ds(s * RPT + k * BK, BK)])


@functools.partial(
    pl.kernel,
    out_type=jax.ShapeDtypeStruct((NC, ACC, D), jnp.float32),
    mesh=_mesh,
    scratch_types=[
        pltpu.VMEM((SB * BK,), jnp.int32),
        pltpu.VMEM((SB, BK), jnp.int32),
        pltpu.VMEM((BK, D), jnp.float32),
        pltpu.VMEM((BK, D), jnp.float32),
        pltpu.VMEM_SHARED((ACC, D), jnp.float32),
        pltpu.SemaphoreType.DMA,
        pltpu.SemaphoreType.DMA,
        pltpu.SemaphoreType.DMA,
    ],
)
def _agg_kernel(g_hbm, src_hbm, dst_hbm, zrow_hbm, out_hbm,
                src_v, dst_v, r0, r1, acc, g0, g1, ssem):
    c = lax.axis_index("c")
    s = lax.axis_index("s")
    base_e = jnp.where(c == 0, s * W0, NS * W0 + s * W1)
    nsb = jnp.where(c == 0, NSB0, NSB1)

    pltpu.sync_copy(zrow_hbm, r0)
    for k in range(RPT // BK):
        pltpu.sync_copy(r0, acc.at[pl.ds(s * RPT + k * BK, BK)])
    plsc.subcore_barrier()

    def outer(o, carry):
        eo = pl.multiple_of(base_e + o * (SB * BK), SB * BK)
        pltpu.sync_copy(src_hbm.at[pl.ds(eo, SB * BK)], src_v)
        pltpu.sync_copy(dst_hbm.at[pl.ds(pl.multiple_of(eo // BK, SB), SB)], dst_v)

        def inner(p, carry2):
            base = p * (2 * BK)
            c0 = pltpu.async_copy(
                g_hbm.at[src_v.at[pl.ds(base, BK)]], r0, g0)
            c1 = pltpu.async_copy(
                g_hbm.at[src_v.at[pl.ds(base + BK, BK)]], r1, g1)
            c0.wait()
            s0 = pltpu.async_copy(r0, acc.at[dst_v.at[2 * p]], ssem, add=True)
            c1.wait()
            s1 = pltpu.async_copy(r1, acc.at[dst_v.at[2 * p + 1]], ssem, add=True)
            s0.wait()
            s1.wait()
            return carry2

        lax.fori_loop(0, SB // 2, inner, 0)
        return carry

    lax.fori_loop(0, nsb, outer, 0)
    plsc.subcore_barrier()

    for k in range(RPT // BK):
        pltpu.sync_copy(acc.at[pl.ds(s * RPT + k * BK, BK)], r0)
        pltpu.sync_copy(r0, out_hbm.at[c, pl.ds(s * RPT + k * BK, BK)])


def _lin_body(deg_ref, x_ref, w_ref, g_ref, dis_ref):
    deg = deg_ref[0, :N, 0:1] + deg_ref[1, :N, 0:1] + 1.0
    dis = lax.rsqrt(deg)
    dis_ref[...] = dis
    h = jnp.dot(x_ref[...], w_ref[...], preferred_element_type=jnp.float32)
    g_ref[...] = h * dis


_lin_call = pl.pallas_call(
    _lin_body,
    out_shape=(jax.ShapeDtypeStruct((N, D), jnp.float32),
               jax.ShapeDtypeStruct((N, 1), jnp.float32)))


def _mid_body(acc_ref, g_ref, dis_ref, b_ref, w_ref, out_ref):
    agg = acc_ref[0, :N, :] + acc_ref[1, :N, :] + g_ref[...]
    h = jax.nn.relu(agg * dis_ref[...] + b_ref[...])
    hw = jnp.dot(h, w_ref[...], preferred_element_type=jnp.float32)
    out_ref[...] = hw * dis_ref[...]


_mid_call = pl.pallas_call(
    _mid_body, out_shape=jax.ShapeDtypeStruct((N, D), jnp.float32))


def _final_body(acc_ref, g_ref, dis_ref, b_ref, out_ref):
    agg = acc_ref[0, :N, :] + acc_ref[1, :N, :] + g_ref[...]
    out_ref[...] = jax.nn.relu(agg * dis_ref[...] + b_ref[...])


_final_call = pl.pallas_call(
    _final_body, out_shape=jax.ShapeDtypeStruct((N, D), jnp.float32))


def kernel(x, edge_index, W1, b1, W2, b2, W3, b3):
    src = edge_index[0]
    dst = edge_index[1]
    pad = ETOT - E
    pad_idx = lax.iota(jnp.int32, pad)
    src_p = jnp.concatenate([src, pad_idx % N])
    dst_p = jnp.concatenate([dst, DUMMY + pad_idx % (ACC - N)])
    dst_deg = dst_p.reshape(NW, NB, BK)
    dst_agg = dst_p.reshape(ETOT // BK, BK)

    zrow = jnp.zeros((BK, D), jnp.float32)
    ones128 = jnp.ones((BK, D), jnp.float32)

    deg_parts = _deg_kernel(dst_deg, ones128, zrow)
    g1, dis_col = _lin_call(deg_parts, x, W1)

    a1 = _agg_kernel(g1, src_p, dst_agg, zrow)
    g2 = _mid_call(a1, g1, dis_col, b1.reshape(1, D), W2)
    a2 = _agg_kernel(g2, src_p, dst_agg, zrow)
    g3 = _mid_call(a2, g2, dis_col, b2.reshape(1, D), W3)
    a3 = _agg_kernel(g3, src_p, dst_agg, zrow)
    return _final_call(a3, g3, dis_col, b3.reshape(1, D))


# speedup vs baseline: 3.3448x; 1.5323x over previous
import functools

import jax
import jax.numpy as jnp
from jax import lax
from jax.experimental import pallas as pl
from jax.experimental.pallas import tpu as pltpu
from jax.experimental.pallas import tpu_sc as plsc

N = 10000
E = 320000
D = 128
NC = 2
NS = 16
NW = NC * NS
BK = 128
SB = 16
NSB = 5
NB = SB * NSB
EPT = NB * BK
NSB0 = 5
NSB1 = 5
W0 = NSB0 * SB * BK
W1 = NSB1 * SB * BK
ETOT = NS * (W0 + W1)
ACC = NS * 640
RPT = ACC // NS
DUMMY = N

_mesh = plsc.VectorSubcoreMesh(
    core_axis_name="c", subcore_axis_name="s", num_cores=NC, num_subcores=NS)


@functools.partial(
    pl.kernel,
    out_type=jax.ShapeDtypeStruct((NC, ACC, D), jnp.float32),
    mesh=_mesh,
    scratch_types=[
        pltpu.VMEM((NB, BK), jnp.int32),
        pltpu.VMEM((BK, D), jnp.float32),
        pltpu.VMEM_SHARED((ACC, D), jnp.float32),
    ],
)
def _deg_kernel(dst_hbm, ones_hbm, zrow_hbm, out_hbm, dst_v, buf, dacc):
    c = lax.axis_index("c")
    s = lax.axis_index("s")
    wid = c * NS + s
    pltpu.sync_copy(dst_hbm.at[wid], dst_v)
    pltpu.sync_copy(zrow_hbm, buf)
    for k in range(RPT // BK):
        pltpu.sync_copy(buf, dacc.at[pl.ds(s * RPT + k * BK, BK)])
    plsc.subcore_barrier()
    pltpu.sync_copy(ones_hbm, buf)

    def body(p, carry):
        pltpu.sync_copy(buf, dacc.at[dst_v.at[p]], add=True)
        return carry

    lax.fori_loop(0, NB, body, 0)
    plsc.subcore_barrier()
    for k in range(RPT // BK):
        pltpu.sync_copy(dacc.at[pl.ds(s * RPT + k * BK, BK)], buf)
        pltpu.sync_copy(buf, out_hbm.at[c, pl.ds(s * RPT + k * BK, BK)])


@functools.partial(
    pl.kernel,
    out_type=jax.ShapeDtypeStruct((NC, ACC, D), jnp.float32),
    mesh=_mesh,
    scratch_types=[
        pltpu.VMEM((SB * BK,), jnp.int32),
        pltpu.VMEM((SB, BK), jnp.int32),
        pltpu.VMEM((BK, D), jnp.float32),
        pltpu.VMEM((BK, D), jnp.float32),
        pltpu.VMEM_SHARED((ACC, D), jnp.float32),
        pltpu.SemaphoreType.DMA,
        pltpu.SemaphoreType.DMA,
        pltpu.SemaphoreType.DMA,
    ],
)
def _agg_kernel(g_hbm, src_hbm, dst_hbm, zrow_hbm, out_hbm,
                src_v, dst_v, r0, r1, acc, g0, g1, ssem):
    c = lax.axis_index("c")
    s = lax.axis_index("s")
    base_e = jnp.where(c == 0, s * W0, NS * W0 + s * W1)
    nsb = jnp.where(c == 0, NSB0, NSB1)

    pltpu.sync_copy(zrow_hbm, r0)
    for k in range(RPT // BK):
        pltpu.sync_copy(r0, acc.at[pl.ds(s * RPT + k * BK, BK)])
    plsc.subcore_barrier()

    def outer(o, carry):
        eo = pl.multiple_of(base_e + o * (SB * BK), SB * BK)
        pltpu.sync_copy(src_hbm.at[pl.ds(eo, SB * BK)], src_v)
        pltpu.sync_copy(dst_hbm.at[pl.ds(pl.multiple_of(eo // BK, SB), SB)], dst_v)

        def inner(p, carry2):
            base = p * (2 * BK)
            c0 = pltpu.async_copy(
                g_hbm.at[src_v.at[pl.ds(base, BK)]], r0, g0)
            c1 = pltpu.async_copy(
                g_hbm.at[src_v.at[pl.ds(base + BK, BK)]], r1, g1)
            c0.wait()
            s0 = pltpu.async_copy(r0, acc.at[dst_v.at[2 * p]], ssem, add=True)
            c1.wait()
            s1 = pltpu.async_copy(r1, acc.at[dst_v.at[2 * p + 1]], ssem, add=True)
            s0.wait()
            s1.wait()
            return carry2

        lax.fori_loop(0, SB // 2, inner, 0)
        return carry

    lax.fori_loop(0, nsb, outer, 0)
    plsc.subcore_barrier()

    for k in range(RPT // BK):
        pltpu.sync_copy(acc.at[pl.ds(s * RPT + k * BK, BK)], r0)
        pltpu.sync_copy(r0, out_hbm.at[c, pl.ds(s * RPT + k * BK, BK)])


def _lin_body(deg_ref, x_ref, w_ref, g_ref, dis_ref):
    deg = deg_ref[0, :N, 0:1] + deg_ref[1, :N, 0:1] + 1.0
    dis = lax.rsqrt(deg)
    dis_ref[...] = dis
    h = jnp.dot(x_ref[...], w_ref[...], preferred_element_type=jnp.float32)
    g_ref[...] = h * dis


_lin_call = pl.pallas_call(
    _lin_body,
    out_shape=(jax.ShapeDtypeStruct((N, D), jnp.float32),
               jax.ShapeDtypeStruct((N, 1), jnp.float32)))


def _mid_body(acc_ref, g_ref, dis_ref, b_ref, w_ref, out_ref):
    agg = acc_ref[0, :N, :] + acc_ref[1, :N, :] + g_ref[...]
    h = jax.nn.relu(agg * dis_ref[...] + b_ref[...])
    hw = jnp.dot(h, w_ref[...], preferred_element_type=jnp.float32)
    out_ref[...] = hw * dis_ref[...]


_mid_call = pl.pallas_call(
    _mid_body, out_shape=jax.ShapeDtypeStruct((N, D), jnp.float32))


def _final_body(acc_ref, g_ref, dis_ref, b_ref, out_ref):
    agg = acc_ref[0, :N, :] + acc_ref[1, :N, :] + g_ref[...]
    out_ref[...] = jax.nn.relu(agg * dis_ref[...] + b_ref[...])


_final_call = pl.pallas_call(
    _final_body, out_shape=jax.ShapeDtypeStruct((N, D), jnp.float32))


def kernel(x, edge_index, W1, b1, W2, b2, W3, b3):
    src = edge_index[0]
    dst = edge_index[1]
    pad = ETOT - E
    pad_idx = lax.iota(jnp.int32, pad)
    src_p = jnp.concatenate([src, pad_idx % N])
    dst_p = jnp.concatenate([dst, DUMMY + pad_idx % (ACC - N)])
    dst_deg = dst_p.reshape(NW, NB, BK)
    dst_agg = dst_p.reshape(ETOT // BK, BK)

    zrow = jnp.zeros((BK, D), jnp.float32)
    ones128 = jnp.ones((BK, D), jnp.float32)

    deg_parts = _deg_kernel(dst_deg, ones128, zrow)
    g1, dis_col = _lin_call(deg_parts, x, W1)

    a1 = _agg_kernel(g1, src_p, dst_agg, zrow)
    g2 = _mid_call(a1, g1, dis_col, b1.reshape(1, D), W2)
    a2 = _agg_kernel(g2, src_p, dst_agg, zrow)
    g3 = _mid_call(a2, g2, dis_col, b2.reshape(1, D), W3)
    a3 = _agg_kernel(g3, src_p, dst_agg, zrow)
    return _final_call(a3, g3, dis_col, b3.reshape(1, D))


# speedup vs baseline: 3.4219x; 1.0230x over previous
import functools

import jax
import jax.numpy as jnp
from jax import lax
from jax.experimental import pallas as pl
from jax.experimental.pallas import tpu as pltpu
from jax.experimental.pallas import tpu_sc as plsc

N = 10000
E = 320000
D = 128
NC = 2
NS = 16
NW = NC * NS
BK = 128
SB = 16
NSB = 5
NB = SB * NSB
EPT = NB * BK
ETOT = NW * EPT
ACC = NS * 640
RPT = ACC // NS
DUMMY = N

_mesh = plsc.VectorSubcoreMesh(
    core_axis_name="c", subcore_axis_name="s", num_cores=NC, num_subcores=NS)


@functools.partial(
    pl.kernel,
    out_type=jax.ShapeDtypeStruct((NC, ACC, D), jnp.float32),
    mesh=_mesh,
    scratch_types=[
        pltpu.VMEM((NB, BK), jnp.int32),
        pltpu.VMEM((BK, D), jnp.float32),
        pltpu.VMEM_SHARED((ACC, D), jnp.float32),
    ],
)
def _deg_kernel(dst_hbm, ones_hbm, zrow_hbm, out_hbm, dst_v, buf, dacc):
    c = lax.axis_index("c")
    s = lax.axis_index("s")
    wid = c * NS + s
    pltpu.sync_copy(dst_hbm.at[wid], dst_v)
    pltpu.sync_copy(zrow_hbm, buf)
    for k in range(RPT // BK):
        pltpu.sync_copy(buf, dacc.at[pl.ds(s * RPT + k * BK, BK)])
    plsc.subcore_barrier()
    pltpu.sync_copy(ones_hbm, buf)

    def body(p, carry):
        pltpu.sync_copy(buf, dacc.at[dst_v.at[p]], add=True)
        return carry

    lax.fori_loop(0, NB, body, 0)
    plsc.subcore_barrier()
    for k in range(RPT // BK):
        pltpu.sync_copy(dacc.at[pl.ds(s * RPT + k * BK, BK)], buf)
        pltpu.sync_copy(buf, out_hbm.at[c, pl.ds(s * RPT + k * BK, BK)])


@functools.partial(
    pl.kernel,
    out_type=jax.ShapeDtypeStruct((NC, ACC, D), jnp.float32),
    mesh=_mesh,
    scratch_types=[
        pltpu.VMEM((SB, BK), jnp.int32),
        pltpu.VMEM((SB, BK), jnp.int32),
        pltpu.VMEM((SB, BK), jnp.int32),
        pltpu.VMEM((SB, BK), jnp.int32),
        pltpu.VMEM((BK, D), jnp.float32),
        pltpu.VMEM((BK, D), jnp.float32),
        pltpu.VMEM_SHARED((ACC, D), jnp.float32),
        pltpu.SemaphoreType.DMA,
        pltpu.SemaphoreType.DMA,
        pltpu.SemaphoreType.DMA,
        pltpu.SemaphoreType.DMA,
    ],
)
def _agg_kernel(g_hbm, src_hbm, dst_hbm, zrow_hbm, out_hbm,
                sv0, sv1, dv0, dv1, r0, r1, acc, g0, g1, ssem, isem):
    c = lax.axis_index("c")
    s = lax.axis_index("s")
    wid = c * NS + s
    ibufs = ((sv0, dv0), (sv1, dv1))

    ip = [pltpu.async_copy(src_hbm.at[wid, 0], sv0, isem),
          pltpu.async_copy(dst_hbm.at[wid, 0], dv0, isem)]

    pltpu.sync_copy(zrow_hbm, r0)
    for k in range(RPT // BK):
        pltpu.sync_copy(r0, acc.at[pl.ds(s * RPT + k * BK, BK)])
    plsc.subcore_barrier()

    for o in range(NSB):
        sv, dv = ibufs[o % 2]
        ip[0].wait()
        ip[1].wait()
        if o + 1 < NSB:
            nsv, ndv = ibufs[(o + 1) % 2]
            ip = [pltpu.async_copy(src_hbm.at[wid, o + 1], nsv, isem),
                  pltpu.async_copy(dst_hbm.at[wid, o + 1], ndv, isem)]

        def inner(p, carry, sv=sv, dv=dv):
            c0 = pltpu.async_copy(
                g_hbm.at[sv.at[2 * p]], r0, g0)
            c1 = pltpu.async_copy(
                g_hbm.at[sv.at[2 * p + 1]], r1, g1)
            c0.wait()
            s0 = pltpu.async_copy(r0, acc.at[dv.at[2 * p]], ssem, add=True)
            c1.wait()
            s1 = pltpu.async_copy(r1, acc.at[dv.at[2 * p + 1]], ssem, add=True)
            s0.wait()
            s1.wait()
            return carry

        lax.fori_loop(0, SB // 2, inner, 0)
    plsc.subcore_barrier()

    for k in range(RPT // BK):
        pltpu.sync_copy(acc.at[pl.ds(s * RPT + k * BK, BK)], r0)
        pltpu.sync_copy(r0, out_hbm.at[c, pl.ds(s * RPT + k * BK, BK)])


def _lin_body(deg_ref, x_ref, w_ref, g_ref, dis_ref):
    deg = deg_ref[0, :N, 0:1] + deg_ref[1, :N, 0:1] + 1.0
    dis = lax.rsqrt(deg)
    dis_ref[...] = dis
    h = jnp.dot(x_ref[...], w_ref[...], preferred_element_type=jnp.float32)
    g_ref[...] = h * dis


_lin_call = pl.pallas_call(
    _lin_body,
    out_shape=(jax.ShapeDtypeStruct((N, D), jnp.float32),
               jax.ShapeDtypeStruct((N, 1), jnp.float32)))


def _mid_body(acc_ref, g_ref, dis_ref, b_ref, w_ref, out_ref):
    agg = acc_ref[0, :N, :] + acc_ref[1, :N, :] + g_ref[...]
    h = jax.nn.relu(agg * dis_ref[...] + b_ref[...])
    hw = jnp.dot(h, w_ref[...], preferred_element_type=jnp.float32)
    out_ref[...] = hw * dis_ref[...]


_mid_call = pl.pallas_call(
    _mid_body, out_shape=jax.ShapeDtypeStruct((N, D), jnp.float32))


def _final_body(acc_ref, g_ref, dis_ref, b_ref, out_ref):
    agg = acc_ref[0, :N, :] + acc_ref[1, :N, :] + g_ref[...]
    out_ref[...] = jax.nn.relu(agg * dis_ref[...] + b_ref[...])


_final_call = pl.pallas_call(
    _final_body, out_shape=jax.ShapeDtypeStruct((N, D), jnp.float32))


def kernel(x, edge_index, W1, b1, W2, b2, W3, b3):
    src = edge_index[0]
    dst = edge_index[1]
    pad = ETOT - E
    pad_idx = lax.iota(jnp.int32, pad)
    src_p = jnp.concatenate([src, pad_idx % N]).reshape(NW, NSB, SB, BK)
    dst_p = jnp.concatenate([dst, DUMMY + pad_idx % (ACC - N)])
    dst_deg = dst_p.reshape(NW, NB, BK)
    dst_agg = dst_p.reshape(NW, NSB, SB, BK)

    zrow = jnp.zeros((BK, D), jnp.float32)
    ones128 = jnp.ones((BK, D), jnp.float32)

    deg_parts = _deg_kernel(dst_deg, ones128, zrow)
    g1, dis_col = _lin_call(deg_parts, x, W1)

    a1 = _agg_kernel(g1, src_p, dst_agg, zrow)
    g2 = _mid_call(a1, g1, dis_col, b1.reshape(1, D), W2)
    a2 = _agg_kernel(g2, src_p, dst_agg, zrow)
    g3 = _mid_call(a2, g2, dis_col, b2.reshape(1, D), W3)
    a3 = _agg_kernel(g3, src_p, dst_agg, zrow)
    return _final_call(a3, g3, dis_col, b3.reshape(1, D))
